# baseline (speedup 1.0000x reference)
import functools

import jax
import jax.numpy as jnp
from jax import lax
from jax.experimental import pallas as pl
from jax.experimental.pallas import tpu as pltpu
from jax.experimental.pallas import tpu_sc as plsc

B = 1024
S = 200
HALF = S // 2
D = 128
NW = 32
SEQ_PER_W = B // NW
NBUF = 3
LANES = 16
D_CHUNKS = D // LANES
ROW_UNROLL = 4


def _emb_body(resp_ref, pos_ref, table_ref, out_ref, pos_v, idx_v,
              rows, gsems, osems):
    wid = lax.axis_index("s") * 2 + lax.axis_index("c")
    wb = wid * SEQ_PER_W

    pltpu.sync_copy(pos_ref, pos_v)
    pltpu.sync_copy(resp_ref.at[0, pl.ds(wb, SEQ_PER_W)], idx_v.at[0])
    pltpu.sync_copy(resp_ref.at[1, pl.ds(wb, SEQ_PER_W)], idx_v.at[1])

    def start_gather(lc, b):
        for h in range(2):
            pltpu.async_copy(table_ref.at[idx_v.at[h, lc]],
                             rows[b].at[pl.ds(h * HALF, HALF)], gsems[b])

    def wait_gather(lc, b):
        for h in range(2):
            pltpu.make_async_copy(table_ref.at[idx_v.at[h, lc]],
                                  rows[b].at[pl.ds(h * HALF, HALF)],
                                  gsems[b]).wait()

    def start_write(lc, b):
        pltpu.async_copy(rows[b], out_ref.at[wb + lc], osems[b])

    def wait_write(lc, b):
        pltpu.make_async_copy(rows[b], out_ref.at[wb + lc], osems[b]).wait()

    def add_rows(b):
        r_v = rows[b]

        def row_body(rr, carry):
            r = rr * ROW_UNROLL
            for u in range(ROW_UNROLL):
                for i in range(D_CHUNKS):
                    sl = pl.ds(i * LANES, LANES)
                    plsc.addupdate(r_v.at[r + u, sl], pos_v[r + u, sl])
            return carry

        lax.fori_loop(0, S // ROW_UNROLL, row_body, None)

    start_gather(0, 0)
    start_gather(1, 1)

    def group_body(g, carry):
        for b in range(NBUF):
            c = NBUF * g + b
            b2 = (b + 2) % NBUF

            @pl.when(c >= 1)
            def _(c=c, b2=b2):
                wait_write(c - 1, b2)

            start_gather(c + 2, b2)

            wait_gather(c, b)
            add_rows(b)
            start_write(c, b)
        return carry

    lax.fori_loop(0, (SEQ_PER_W - 2) // NBUF, group_body, None)

    for c, b in ((SEQ_PER_W - 2, 0), (SEQ_PER_W - 1, 1)):
        wait_write(c - 1, (b + 2) % NBUF)
        wait_gather(c, b)
        add_rows(b)
        start_write(c, b)
    wait_write(SEQ_PER_W - 1, 1)


@jax.jit
def _emb(resp, position_table, response_table):
    mesh = plsc.VectorSubcoreMesh(core_axis_name="c", subcore_axis_name="s")
    kfn = functools.partial(
        pl.kernel,
        out_type=jax.ShapeDtypeStruct((B, S, D), jnp.float32),
        mesh=mesh,
        scratch_types=[
            pltpu.VMEM((S, D), jnp.float32),
            pltpu.VMEM((2, SEQ_PER_W, HALF), jnp.int32),
            tuple(pltpu.VMEM((S, D), jnp.float32) for _ in range(NBUF)),
            tuple(pltpu.SemaphoreType.DMA for _ in range(NBUF)),
            tuple(pltpu.SemaphoreType.DMA for _ in range(NBUF)),
        ],
    )(_emb_body)
    return kfn(resp, position_table, response_table)


def kernel(response, position_table, response_table):
    resp = (response.astype(jnp.int32)
            .reshape(B, 2, HALF).transpose(1, 0, 2))
    return _emb(resp, position_table, response_table)


# speedup vs baseline: 1.1673x; 1.1673x over previous
import functools

import jax
import jax.numpy as jnp
from jax import lax
from jax.experimental import pallas as pl
from jax.experimental.pallas import tpu as pltpu
from jax.experimental.pallas import tpu_sc as plsc

B = 1024
S = 200
HALF = S // 2
D = 128
NW = 32
SEQ_PER_W = B // NW
NBUF = 4
LANES = 16
D_CHUNKS = D // LANES
ROW_UNROLL = 4


def _emb_body(resp_ref, pos_ref, table_ref, out_ref, pos_v, ibuf,
              rows, gsems, osems, isems):
    wid = lax.axis_index("s") * 2 + lax.axis_index("c")
    wb = wid * SEQ_PER_W

    pltpu.sync_copy(pos_ref, pos_v)

    def start_idx(lc, k):
        pltpu.async_copy(resp_ref.at[wb + lc], ibuf.at[k], isems[k])

    def wait_idx(lc, k):
        pltpu.make_async_copy(resp_ref.at[wb + lc], ibuf.at[k],
                              isems[k]).wait()

    def start_gather(lc, k, b):
        for h in range(2):
            pltpu.async_copy(table_ref.at[ibuf.at[k, h]],
                             rows[b].at[pl.ds(h * HALF, HALF)], gsems[b])

    def wait_gather(lc, k, b):
        for h in range(2):
            pltpu.make_async_copy(table_ref.at[ibuf.at[k, h]],
                                  rows[b].at[pl.ds(h * HALF, HALF)],
                                  gsems[b]).wait()

    def start_write(lc, b):
        pltpu.async_copy(rows[b], out_ref.at[wb + lc], osems[b])

    def wait_write(lc, b):
        pltpu.make_async_copy(rows[b], out_ref.at[wb + lc], osems[b]).wait()

    def add_rows(b):
        r_v = rows[b]

        def row_body(rr, carry):
            r = rr * ROW_UNROLL
            for u in range(ROW_UNROLL):
                for i in range(D_CHUNKS):
                    sl = pl.ds(i * LANES, LANES)
                    plsc.addupdate(r_v.at[r + u, sl], pos_v[r + u, sl])
            return carry

        lax.fori_loop(0, S // ROW_UNROLL, row_body, None)

    start_idx(0, 0)
    start_idx(1, 1)
    start_idx(2, 2)
    wait_idx(0, 0)
    start_gather(0, 0, 0)
    wait_idx(1, 1)
    start_gather(1, 1, 1)

    def group_body(g, carry):
        for b in range(NBUF):
            c = NBUF * g + b
            b2 = (b + 2) % NBUF

            @pl.when(c + 3 < SEQ_PER_W)
            def _(c=c, b=b):
                start_idx(c + 3, (b + 3) % NBUF)

            @pl.when(c + 2 < SEQ_PER_W)
            def _(c=c, b2=b2):
                @pl.when(c >= 2)
                def _():
                    wait_write(c - 2, b2)
                wait_idx(c + 2, b2)
                start_gather(c + 2, b2, b2)

            wait_gather(c, b, b)
            add_rows(b)
            start_write(c, b)
        return carry

    lax.fori_loop(0, SEQ_PER_W // NBUF, group_body, None)

    wait_write(SEQ_PER_W - 2, 2)
    wait_write(SEQ_PER_W - 1, 3)


@jax.jit
def _emb(resp, position_table, response_table):
    mesh = plsc.VectorSubcoreMesh(core_axis_name="c", subcore_axis_name="s")
    kfn = functools.partial(
        pl.kernel,
        out_type=jax.ShapeDtypeStruct((B, S, D), jnp.float32),
        mesh=mesh,
        scratch_types=[
            pltpu.VMEM((S, D), jnp.float32),
            pltpu.VMEM((NBUF, 2, HALF), jnp.int32),
            tuple(pltpu.VMEM((S, D), jnp.float32) for _ in range(NBUF)),
            tuple(pltpu.SemaphoreType.DMA for _ in range(NBUF)),
            tuple(pltpu.SemaphoreType.DMA for _ in range(NBUF)),
            tuple(pltpu.SemaphoreType.DMA for _ in range(NBUF)),
        ],
    )(_emb_body)
    return kfn(resp, position_table, response_table)


def kernel(response, position_table, response_table):
    resp = response.astype(jnp.int32).reshape(B, 2, HALF)
    return _emb(resp, position_table, response_table)


# speedup vs baseline: 1.1924x; 1.0215x over previous
import functools

import jax
import jax.numpy as jnp
from jax import lax
from jax.experimental import pallas as pl
from jax.experimental.pallas import tpu as pltpu
from jax.experimental.pallas import tpu_sc as plsc

B = 1024
S = 200
HALF = S // 2
D = 128
NW = 32
SEQ_PER_W = B // NW
NBUF = 4
LANES = 16
D_CHUNKS = D // LANES
ROW_UNROLL = 4


def _emb_body(resp_ref, pos_ref, table_ref, out_ref, pos_v, ibuf,
              rows, gsems, osems, isems, psem):
    wid = lax.axis_index("s") * 2 + lax.axis_index("c")
    wb = wid * SEQ_PER_W

    def start_idx(lc, k):
        pltpu.async_copy(resp_ref.at[wb + lc], ibuf.at[k], isems[k])

    def wait_idx(lc, k):
        pltpu.make_async_copy(resp_ref.at[wb + lc], ibuf.at[k],
                              isems[k]).wait()

    def start_gather(lc, k, b):
        for h in range(2):
            pltpu.async_copy(table_ref.at[ibuf.at[k, h]],
                             rows[b].at[pl.ds(h * HALF, HALF)], gsems[b])

    def wait_gather(lc, k, b):
        for h in range(2):
            pltpu.make_async_copy(table_ref.at[ibuf.at[k, h]],
                                  rows[b].at[pl.ds(h * HALF, HALF)],
                                  gsems[b]).wait()

    def start_write(lc, b):
        pltpu.async_copy(rows[b], out_ref.at[wb + lc], osems[b])

    def wait_write(lc, b):
        pltpu.make_async_copy(rows[b], out_ref.at[wb + lc], osems[b]).wait()

    def add_rows(b):
        r_v = rows[b]

        def row_body(rr, carry):
            r = rr * ROW_UNROLL
            for u in range(ROW_UNROLL):
                for i in range(D_CHUNKS):
                    sl = pl.ds(i * LANES, LANES)
                    plsc.addupdate(r_v.at[r + u, sl], pos_v[r + u, sl])
            return carry

        lax.fori_loop(0, S // ROW_UNROLL, row_body, None)

    start_idx(0, 0)
    start_idx(1, 1)
    start_idx(2, 2)
    pltpu.async_copy(pos_ref, pos_v, psem)
    wait_idx(0, 0)
    start_gather(0, 0, 0)
    wait_idx(1, 1)
    start_gather(1, 1, 1)
    pltpu.make_async_copy(pos_ref, pos_v, psem).wait()

    def group_body(g, carry):
        for b in range(NBUF):
            c = NBUF * g + b
            b2 = (b + 2) % NBUF

            @pl.when(c + 3 < SEQ_PER_W)
            def _(c=c, b=b):
                start_idx(c + 3, (b + 3) % NBUF)

            @pl.when(c + 2 < SEQ_PER_W)
            def _(c=c, b2=b2):
                @pl.when(c >= 2)
                def _():
                    wait_write(c - 2, b2)
                wait_idx(c + 2, b2)
                start_gather(c + 2, b2, b2)

            wait_gather(c, b, b)
            add_rows(b)
            start_write(c, b)
        return carry

    lax.fori_loop(0, SEQ_PER_W // NBUF, group_body, None)

    wait_write(SEQ_PER_W - 2, 2)
    wait_write(SEQ_PER_W - 1, 3)


@jax.jit
def _emb(resp, position_table, response_table):
    mesh = plsc.VectorSubcoreMesh(core_axis_name="c", subcore_axis_name="s")
    kfn = functools.partial(
        pl.kernel,
        out_type=jax.ShapeDtypeStruct((B, S, D), jnp.float32),
        mesh=mesh,
        scratch_types=[
            pltpu.VMEM((S, D), jnp.float32),
            pltpu.VMEM((NBUF, 2, HALF), jnp.int32),
            tuple(pltpu.VMEM((S, D), jnp.float32) for _ in range(NBUF)),
            tuple(pltpu.SemaphoreType.DMA for _ in range(NBUF)),
            tuple(pltpu.SemaphoreType.DMA for _ in range(NBUF)),
            tuple(pltpu.SemaphoreType.DMA for _ in range(NBUF)),
            pltpu.SemaphoreType.DMA,
        ],
    )(_emb_body)
    return kfn(resp, position_table, response_table)


def kernel(response, position_table, response_table):
    resp = response.astype(jnp.int32).reshape(B, 2, HALF)
    return _emb(resp, position_table, response_table)
